# baseline (speedup 1.0000x reference)
import functools

import jax
import jax.numpy as jnp
from jax.experimental import pallas as pl
from jax.experimental.pallas import tpu as pltpu

_NATIVE = (((1,), (0,)), ((), ()))
_LHS_T = (((0,), (0,)), ((), ()))


def _pack_body(a_ref, cs_ref, ab_ref, *, n, bs):
    k = pl.program_id(1)
    a = a_ref[...]
    rows = jax.lax.broadcasted_iota(jnp.int32, a.shape, 0)
    b = jnp.where(rows < n - k * bs, a, 0.0)
    ab_ref[...] = b.astype(jnp.int8)
    ones8 = jnp.ones((8, bs), jnp.float32)
    s = jax.lax.dot_general(ones8, b, _NATIVE,
                            preferred_element_type=jnp.float32)

    @pl.when(k == 0)
    def _init():
        cs_ref[...] = jnp.zeros_like(cs_ref)

    cs_ref[...] += s


def _dinv_row(cs_ref):
    return jax.lax.rsqrt(cs_ref[:1, :] + 1.0)


def _xpanel(x_ref, cs_ref, base, n):
    xt = jnp.transpose(x_ref[...])
    ys = xt * _dinv_row(cs_ref)
    colg = base + jax.lax.broadcasted_iota(jnp.int32, ys.shape, 1)
    return jnp.where(colg < n, ys, 0.0).astype(jnp.bfloat16)


def _layer1_body(ab_ref, xs_ref, xd_ref, css_ref, csd_ref, w1_ref, b1_ref,
                 w2p_ref, pa_ref, out_ref, acc_ref, *, n, bs, bd, nk):
    i = pl.program_id(0)
    k = pl.program_id(1)
    ys = _xpanel(xs_ref, css_ref, k * bs, n)

    @pl.when(k == 0)
    def _init():
        acc_ref[...] = jnp.zeros_like(acc_ref)

    acc_ref[...] += jax.lax.dot_general(
        ys, ab_ref[...].astype(jnp.bfloat16), _NATIVE,
        preferred_element_type=jnp.float32)

    @pl.when(k == nk - 1)
    def _epilogue():
        yd = _xpanel(xd_ref, csd_ref, i * bd, n)
        vt = acc_ref[...] + yd.astype(jnp.float32)
        dinv = _dinv_row(csd_ref)
        ht = jax.lax.dot_general(w1_ref[...], vt, _LHS_T,
                                 preferred_element_type=jnp.float32)
        ht = ht * dinv + b1_ref[...]
        pa = pa_ref[0]
        ht = jnp.where(ht >= 0, ht, pa * ht)
        y2t = jax.lax.dot_general(w2p_ref[...], ht, _LHS_T,
                                  preferred_element_type=jnp.float32) * dinv
        colg = i * bd + jax.lax.broadcasted_iota(jnp.int32, y2t.shape, 1)
        out_ref[...] = jnp.where(colg < n, y2t, 0.0).astype(jnp.bfloat16)


def _layer2_body(ab_ref, ys_ref, yd_ref, csd_ref, b2_ref, out_ref, acc_ref,
                 *, c, nk):
    k = pl.program_id(1)

    @pl.when(k == 0)
    def _init():
        acc_ref[...] = jnp.zeros_like(acc_ref)

    acc_ref[...] += jax.lax.dot_general(
        ys_ref[...], ab_ref[...].astype(jnp.bfloat16), _NATIVE,
        preferred_element_type=jnp.float32)

    @pl.when(k == nk - 1)
    def _epilogue():
        dinv = _dinv_row(csd_ref)
        ut = dinv * (acc_ref[...] + yd_ref[...].astype(jnp.float32)) + b2_ref[...]
        rowv = jax.lax.broadcasted_iota(jnp.int32, ut.shape, 0) < c
        m = jnp.max(jnp.where(rowv, ut, jnp.float32(-1e30)), axis=0,
                    keepdims=True)
        e = jnp.where(rowv, jnp.exp(ut - m), 0.0)
        lse = jnp.log(jnp.sum(e, axis=0, keepdims=True))
        out_ref[...] = ut - m - lse


def kernel(x, A, W1, b1, prelu_a, W2, b2):
    n, din = x.shape
    hdim = W1.shape[1]
    c = W2.shape[1]
    f = 128
    assert din == f

    bd = min(1024, max(128, ((n + 127) // 128) * 128))
    bs = bd
    nd = (n + bd - 1) // bd
    nk = (n + bs - 1) // bs
    npad = nd * bd

    fp = jnp.float32
    bf = jnp.bfloat16
    params = pltpu.CompilerParams(
        dimension_semantics=("parallel", "arbitrary"))

    a_spec = pl.BlockSpec((bs, bd), lambda i, k: (k, i))

    colsum, ab = pl.pallas_call(
        functools.partial(_pack_body, n=n, bs=bs),
        grid=(nd, nk),
        in_specs=[a_spec],
        out_specs=[pl.BlockSpec((8, bd), lambda i, k: (0, i)), a_spec],
        out_shape=[jax.ShapeDtypeStruct((8, npad), fp),
                   jax.ShapeDtypeStruct((npad, npad), jnp.int8)],
        compiler_params=params,
    )(A)

    w2p = jnp.pad(W2, ((0, 0), (0, f - c)))
    b1c = b1.reshape(hdim, 1)
    b2c = jnp.pad(b2, (0, f - c)).reshape(f, 1)

    xs_spec = pl.BlockSpec((bs, f), lambda i, k: (k, 0))
    xd_spec = pl.BlockSpec((bd, f), lambda i, k: (i, 0))
    css_spec = pl.BlockSpec((8, bs), lambda i, k: (0, k))
    csd_spec = pl.BlockSpec((8, bd), lambda i, k: (0, i))
    ys_spec = pl.BlockSpec((f, bs), lambda i, k: (0, k))
    yd_spec = pl.BlockSpec((f, bd), lambda i, k: (0, i))
    full = lambda shape: pl.BlockSpec(shape, lambda i, k: (0, 0))
    out_spec = pl.BlockSpec((f, bd), lambda i, k: (0, i))
    acc = pltpu.VMEM((f, bd), fp)

    y2t = pl.pallas_call(
        functools.partial(_layer1_body, n=n, bs=bs, bd=bd, nk=nk),
        grid=(nd, nk),
        in_specs=[a_spec, xs_spec, xd_spec, css_spec, csd_spec,
                  full((din, hdim)), full((hdim, 1)), full((hdim, f)),
                  pl.BlockSpec(memory_space=pltpu.SMEM)],
        out_specs=out_spec,
        out_shape=jax.ShapeDtypeStruct((f, npad), bf),
        scratch_shapes=[acc],
        compiler_params=params,
    )(ab, x, x, colsum, colsum, W1, b1c, w2p, prelu_a.reshape(1))

    outt = pl.pallas_call(
        functools.partial(_layer2_body, c=c, nk=nk),
        grid=(nd, nk),
        in_specs=[a_spec, ys_spec, yd_spec, csd_spec, full((f, 1))],
        out_specs=out_spec,
        out_shape=jax.ShapeDtypeStruct((f, npad), fp),
        scratch_shapes=[acc],
        compiler_params=params,
    )(ab, y2t, y2t, colsum, b2c)

    return outt[:c, :n].T


# speedup vs baseline: 1.4395x; 1.4395x over previous
import functools

import jax
import jax.numpy as jnp
from jax.experimental import pallas as pl
from jax.experimental.pallas import tpu as pltpu

_NATIVE = (((1,), (0,)), ((), ()))


def _pack_body(a_ref, cs_ref, ab_ref, *, n, bs):
    k = pl.program_id(1)
    a = a_ref[...]
    rows = jax.lax.broadcasted_iota(jnp.int32, a.shape, 0)
    b = jnp.where(rows < n - k * bs, a, 0.0)
    ab_ref[...] = b.astype(jnp.int8)
    ones8 = jnp.ones((8, bs), jnp.float32)
    s = jax.lax.dot_general(ones8, b, _NATIVE,
                            preferred_element_type=jnp.float32)

    @pl.when(k == 0)
    def _init():
        cs_ref[...] = jnp.zeros_like(cs_ref)

    cs_ref[...] += s


def _agg_block(ys_ref, ab_ref):
    return jax.lax.dot_general(
        ys_ref[...], ab_ref[...].astype(jnp.bfloat16), _NATIVE,
        preferred_element_type=jnp.float32)


def _layer1_body(ab_ref, ys_ref, yd_ref, dinv_ref, w1t_ref, b1_ref, w2t_ref,
                 pa_ref, out_ref, acc_ref, *, n, bd, nk):
    i = pl.program_id(0)
    k = pl.program_id(1)

    @pl.when(k == 0)
    def _init():
        acc_ref[...] = jnp.zeros_like(acc_ref)

    acc_ref[...] += _agg_block(ys_ref, ab_ref)

    @pl.when(k == nk - 1)
    def _epilogue():
        vt = acc_ref[...] + yd_ref[...].astype(jnp.float32)
        dinv = dinv_ref[:1, :]
        ht = jax.lax.dot_general(w1t_ref[...], vt, _NATIVE,
                                 preferred_element_type=jnp.float32)
        ht = ht * dinv + b1_ref[...]
        pa = pa_ref[0]
        ht = jnp.where(ht >= 0, ht, pa * ht)
        y2t = jax.lax.dot_general(w2t_ref[...], ht, _NATIVE,
                                  preferred_element_type=jnp.float32) * dinv
        colg = i * bd + jax.lax.broadcasted_iota(jnp.int32, y2t.shape, 1)
        out_ref[...] = jnp.where(colg < n, y2t, 0.0).astype(jnp.bfloat16)


def _layer2_body(ab_ref, ys_ref, yd_ref, dinv_ref, b2_ref, out_ref, acc_ref,
                 *, c, nk):
    k = pl.program_id(1)

    @pl.when(k == 0)
    def _init():
        acc_ref[...] = jnp.zeros_like(acc_ref)

    acc_ref[...] += _agg_block(ys_ref, ab_ref)

    @pl.when(k == nk - 1)
    def _epilogue():
        dinv = dinv_ref[:1, :]
        ut = dinv * (acc_ref[...] + yd_ref[...].astype(jnp.float32)) + b2_ref[...]
        rowv = jax.lax.broadcasted_iota(jnp.int32, ut.shape, 0) < c
        m = jnp.max(jnp.where(rowv, ut, jnp.float32(-1e30)), axis=0,
                    keepdims=True)
        e = jnp.where(rowv, jnp.exp(ut - m), 0.0)
        lse = jnp.log(jnp.sum(e, axis=0, keepdims=True))
        out_ref[...] = ut - m - lse


def kernel(x, A, W1, b1, prelu_a, W2, b2):
    n, din = x.shape
    hdim = W1.shape[1]
    c = W2.shape[1]
    f = 128
    assert din == f

    bd = min(2048, max(128, ((n + 127) // 128) * 128))
    bs = bd
    nd = (n + bd - 1) // bd
    nk = (n + bs - 1) // bs
    npad = nd * bd

    fp = jnp.float32
    bf = jnp.bfloat16
    params = pltpu.CompilerParams(
        dimension_semantics=("parallel", "arbitrary"))

    a_spec = pl.BlockSpec((bs, bd), lambda i, k: (k, i))

    colsum, ab = pl.pallas_call(
        functools.partial(_pack_body, n=n, bs=bs),
        grid=(nd, nk),
        in_specs=[a_spec],
        out_specs=[pl.BlockSpec((8, bd), lambda i, k: (0, i)), a_spec],
        out_shape=[jax.ShapeDtypeStruct((8, npad), fp),
                   jax.ShapeDtypeStruct((npad, npad), jnp.int8)],
        compiler_params=params,
    )(A)

    deg = colsum[0, :n] + 1.0
    dinv = deg ** -0.5
    pad_c = ((0, 0), (0, npad - n))
    y1t = jnp.pad((dinv[None, :] * x.T).astype(bf), pad_c)
    dinv8 = jnp.pad(jnp.broadcast_to(dinv[None, :], (8, n)), pad_c)
    w1t = W1.T
    w2t = jnp.pad(W2, ((0, 0), (0, f - c))).T
    b1c = b1.reshape(hdim, 1)
    b2c = jnp.pad(b2, (0, f - c)).reshape(f, 1)

    ys_spec = pl.BlockSpec((f, bs), lambda i, k: (0, k))
    yd_spec = pl.BlockSpec((f, bd), lambda i, k: (0, i))
    d8_spec = pl.BlockSpec((8, bd), lambda i, k: (0, i))
    full = lambda shape: pl.BlockSpec(shape, lambda i, k: (0, 0))
    out_spec = pl.BlockSpec((f, bd), lambda i, k: (0, i))
    acc = pltpu.VMEM((f, bd), fp)

    y2t = pl.pallas_call(
        functools.partial(_layer1_body, n=n, bd=bd, nk=nk),
        grid=(nd, nk),
        in_specs=[a_spec, ys_spec, yd_spec, d8_spec,
                  full((hdim, f)), full((hdim, 1)), full((f, hdim)),
                  pl.BlockSpec(memory_space=pltpu.SMEM)],
        out_specs=out_spec,
        out_shape=jax.ShapeDtypeStruct((f, npad), bf),
        scratch_shapes=[acc],
        compiler_params=params,
    )(ab, y1t, y1t, dinv8, w1t, b1c, w2t, prelu_a.reshape(1))

    outt = pl.pallas_call(
        functools.partial(_layer2_body, c=c, nk=nk),
        grid=(nd, nk),
        in_specs=[a_spec, ys_spec, yd_spec, d8_spec, full((f, 1))],
        out_specs=out_spec,
        out_shape=jax.ShapeDtypeStruct((f, npad), fp),
        scratch_shapes=[acc],
        compiler_params=params,
    )(ab, y2t, y2t, dinv8, b2c)

    return outt[:c, :n].T


# speedup vs baseline: 1.4945x; 1.0382x over previous
import functools

import jax
import jax.numpy as jnp
from jax.experimental import pallas as pl
from jax.experimental.pallas import tpu as pltpu

_NATIVE = (((1,), (0,)), ((), ()))


def _pack_body(a_ref, cs_ref, ab_ref, *, n, bs):
    k = pl.program_id(1)
    a = a_ref[...]
    rows = jax.lax.broadcasted_iota(jnp.int32, a.shape, 0)
    b = jnp.where(rows < n - k * bs, a, 0.0)
    ab_ref[...] = b.astype(jnp.int8)
    ones8 = jnp.ones((8, bs), jnp.float32)
    s = jax.lax.dot_general(ones8, b, _NATIVE,
                            preferred_element_type=jnp.float32)

    @pl.when(k == 0)
    def _init():
        cs_ref[...] = jnp.zeros_like(cs_ref)

    cs_ref[...] += s


def _agg_block(ys_ref, ab_ref):
    return jax.lax.dot_general(
        ys_ref[...], ab_ref[...].astype(jnp.bfloat16), _NATIVE,
        preferred_element_type=jnp.float32)


def _layer1_body(ab_ref, ys_ref, yd_ref, dinv_ref, w1t_ref, b1_ref, w2t_ref,
                 pa_ref, out_ref, acc_ref, *, n, bd, nk):
    i = pl.program_id(0)
    k = pl.program_id(1)

    @pl.when(k == 0)
    def _init():
        acc_ref[...] = jnp.zeros_like(acc_ref)

    acc_ref[...] += _agg_block(ys_ref, ab_ref)

    @pl.when(k == nk - 1)
    def _epilogue():
        vt = acc_ref[...] + yd_ref[...].astype(jnp.float32)
        dinv = dinv_ref[:1, :]
        ht = jax.lax.dot_general(w1t_ref[...], vt, _NATIVE,
                                 preferred_element_type=jnp.float32)
        ht = ht * dinv + b1_ref[...]
        pa = pa_ref[0]
        ht = jnp.where(ht >= 0, ht, pa * ht)
        y2t = jax.lax.dot_general(w2t_ref[...], ht, _NATIVE,
                                  preferred_element_type=jnp.float32) * dinv
        colg = i * bd + jax.lax.broadcasted_iota(jnp.int32, y2t.shape, 1)
        out_ref[...] = jnp.where(colg < n, y2t, 0.0).astype(jnp.bfloat16)


def _layer2_body(ab_ref, ys_ref, yd_ref, dinv_ref, b2_ref, out_ref, acc_ref,
                 *, c, nk):
    k = pl.program_id(1)

    @pl.when(k == 0)
    def _init():
        acc_ref[...] = jnp.zeros_like(acc_ref)

    acc_ref[...] += _agg_block(ys_ref, ab_ref)

    @pl.when(k == nk - 1)
    def _epilogue():
        dinv = dinv_ref[:1, :]
        ut = dinv * (acc_ref[...] + yd_ref[...].astype(jnp.float32)) + b2_ref[...]
        rowv = jax.lax.broadcasted_iota(jnp.int32, ut.shape, 0) < c
        m = jnp.max(jnp.where(rowv, ut, jnp.float32(-1e30)), axis=0,
                    keepdims=True)
        e = jnp.where(rowv, jnp.exp(ut - m), 0.0)
        lse = jnp.log(jnp.sum(e, axis=0, keepdims=True))
        out_ref[...] = ut - m - lse


def kernel(x, A, W1, b1, prelu_a, W2, b2):
    n, din = x.shape
    hdim = W1.shape[1]
    c = W2.shape[1]
    f = 128
    assert din == f

    npad128 = ((n + 127) // 128) * 128
    if npad128 <= 2560:
        npad = npad128
        blk1 = blk2 = npad
    else:
        npad = ((n + 10239) // 10240) * 10240
        blk1, blk2 = 2048, 2560
    nd1 = nk1 = npad // blk1
    nd = nk = npad // blk2
    bd = bs = blk2

    fp = jnp.float32
    bf = jnp.bfloat16
    params = pltpu.CompilerParams(
        dimension_semantics=("parallel", "arbitrary"))

    a1_spec = pl.BlockSpec((blk1, blk1), lambda i, k: (k, i))
    a_spec = pl.BlockSpec((bs, bd), lambda i, k: (k, i))

    colsum, ab = pl.pallas_call(
        functools.partial(_pack_body, n=n, bs=blk1),
        grid=(nd1, nk1),
        in_specs=[a1_spec],
        out_specs=[pl.BlockSpec((8, blk1), lambda i, k: (0, i)), a1_spec],
        out_shape=[jax.ShapeDtypeStruct((8, npad), fp),
                   jax.ShapeDtypeStruct((npad, npad), jnp.int8)],
        compiler_params=params,
    )(A)

    deg = colsum[0, :n] + 1.0
    dinv = deg ** -0.5
    pad_c = ((0, 0), (0, npad - n))
    y1t = jnp.pad((dinv[None, :] * x.T).astype(bf), pad_c)
    dinv8 = jnp.pad(jnp.broadcast_to(dinv[None, :], (8, n)), pad_c)
    w1t = W1.T
    w2t = jnp.pad(W2, ((0, 0), (0, f - c))).T
    b1c = b1.reshape(hdim, 1)
    b2c = jnp.pad(b2, (0, f - c)).reshape(f, 1)

    ys_spec = pl.BlockSpec((f, bs), lambda i, k: (0, k))
    yd_spec = pl.BlockSpec((f, bd), lambda i, k: (0, i))
    d8_spec = pl.BlockSpec((8, bd), lambda i, k: (0, i))
    full = lambda shape: pl.BlockSpec(shape, lambda i, k: (0, 0))
    out_spec = pl.BlockSpec((f, bd), lambda i, k: (0, i))
    acc = pltpu.VMEM((f, bd), fp)

    y2t = pl.pallas_call(
        functools.partial(_layer1_body, n=n, bd=bd, nk=nk),
        grid=(nd, nk),
        in_specs=[a_spec, ys_spec, yd_spec, d8_spec,
                  full((hdim, f)), full((hdim, 1)), full((f, hdim)),
                  pl.BlockSpec(memory_space=pltpu.SMEM)],
        out_specs=out_spec,
        out_shape=jax.ShapeDtypeStruct((f, npad), bf),
        scratch_shapes=[acc],
        compiler_params=params,
    )(ab, y1t, y1t, dinv8, w1t, b1c, w2t, prelu_a.reshape(1))

    outt = pl.pallas_call(
        functools.partial(_layer2_body, c=c, nk=nk),
        grid=(nd, nk),
        in_specs=[a_spec, ys_spec, yd_spec, d8_spec, full((f, 1))],
        out_specs=out_spec,
        out_shape=jax.ShapeDtypeStruct((f, npad), fp),
        scratch_shapes=[acc],
        compiler_params=params,
    )(ab, y2t, y2t, dinv8, b2c)

    return outt[:c, :n].T


# speedup vs baseline: 1.5273x; 1.0219x over previous
import functools

import jax
import jax.numpy as jnp
from jax.experimental import pallas as pl
from jax.experimental.pallas import tpu as pltpu

_NATIVE = (((1,), (0,)), ((), ()))


def _pack_body(a_ref, cs_ref, ab_ref, *, n, bs):
    k = pl.program_id(1)
    a = a_ref[...]
    rows = jax.lax.broadcasted_iota(jnp.int32, a.shape, 0)
    b = jnp.where(rows < n - k * bs, a, 0.0)
    ab_ref[...] = b.astype(jnp.int8)
    ones8 = jnp.ones((8, bs), jnp.float32)
    s = jax.lax.dot_general(ones8, b, _NATIVE,
                            preferred_element_type=jnp.float32)

    @pl.when(k == 0)
    def _init():
        cs_ref[...] = jnp.zeros_like(cs_ref)

    cs_ref[...] += s


def _agg_block(ys_ref, ab_ref):
    return jax.lax.dot_general(
        ys_ref[...], ab_ref[...].astype(jnp.bfloat16), _NATIVE,
        preferred_element_type=jnp.float32)


def _layer1_body(ab_ref, ys_ref, yd_ref, dinv_ref, w1t_ref, b1_ref, w2t_ref,
                 pa_ref, out_ref, acc_ref, *, n, bd, nk):
    i = pl.program_id(0)
    k = pl.program_id(1)

    @pl.when(k == 0)
    def _init():
        acc_ref[...] = jnp.zeros_like(acc_ref)

    acc_ref[...] += _agg_block(ys_ref, ab_ref)

    @pl.when(k == nk - 1)
    def _epilogue():
        vt = acc_ref[...] + yd_ref[...].astype(jnp.float32)
        dinv = dinv_ref[:1, :]
        ht = jax.lax.dot_general(w1t_ref[...], vt, _NATIVE,
                                 preferred_element_type=jnp.float32)
        ht = ht * dinv + b1_ref[...]
        pa = pa_ref[0]
        ht = jnp.where(ht >= 0, ht, pa * ht)
        y2t = jax.lax.dot_general(w2t_ref[...], ht, _NATIVE,
                                  preferred_element_type=jnp.float32) * dinv
        colg = i * bd + jax.lax.broadcasted_iota(jnp.int32, y2t.shape, 1)
        out_ref[...] = jnp.where(colg < n, y2t, 0.0).astype(jnp.bfloat16)


def _layer2_body(ab_ref, ys_ref, yd_ref, dinv_ref, b2_ref, out_ref, acc_ref,
                 *, c, nk):
    k = pl.program_id(1)

    @pl.when(k == 0)
    def _init():
        acc_ref[...] = jnp.zeros_like(acc_ref)

    acc_ref[...] += _agg_block(ys_ref, ab_ref)

    @pl.when(k == nk - 1)
    def _epilogue():
        dinv = dinv_ref[:1, :]
        ut = dinv * (acc_ref[...] + yd_ref[...].astype(jnp.float32)) + b2_ref[...]
        rowv = jax.lax.broadcasted_iota(jnp.int32, ut.shape, 0) < c
        m = jnp.max(jnp.where(rowv, ut, jnp.float32(-1e30)), axis=0,
                    keepdims=True)
        e = jnp.where(rowv, jnp.exp(ut - m), 0.0)
        lse = jnp.log(jnp.sum(e, axis=0, keepdims=True))
        out_ref[...] = ut - m - lse


def kernel(x, A, W1, b1, prelu_a, W2, b2):
    n, din = x.shape
    hdim = W1.shape[1]
    c = W2.shape[1]
    f = 128
    assert din == f

    npad128 = ((n + 127) // 128) * 128
    if npad128 <= 2560:
        npad = npad128
        bs1 = bd1 = bs = bd = npad
    else:
        npad = ((n + 10239) // 10240) * 10240
        bs1, bd1 = 2048, 2560
        bs, bd = 5120, 2560
    nd1, nk1 = npad // bd1, npad // bs1
    nd, nk = npad // bd, npad // bs

    fp = jnp.float32
    bf = jnp.bfloat16
    params = pltpu.CompilerParams(
        dimension_semantics=("parallel", "arbitrary"))

    a1_spec = pl.BlockSpec((bs1, bd1), lambda i, k: (k, i))
    a_spec = pl.BlockSpec((bs, bd), lambda i, k: (k, i))

    colsum, ab = pl.pallas_call(
        functools.partial(_pack_body, n=n, bs=bs1),
        grid=(nd1, nk1),
        in_specs=[a1_spec],
        out_specs=[pl.BlockSpec((8, bd1), lambda i, k: (0, i)), a1_spec],
        out_shape=[jax.ShapeDtypeStruct((8, npad), fp),
                   jax.ShapeDtypeStruct((npad, npad), jnp.int8)],
        compiler_params=params,
    )(A)

    deg = colsum[0, :n] + 1.0
    dinv = deg ** -0.5
    pad_c = ((0, 0), (0, npad - n))
    y1t = jnp.pad((dinv[None, :] * x.T).astype(bf), pad_c)
    dinv8 = jnp.pad(jnp.broadcast_to(dinv[None, :], (8, n)), pad_c)
    w1t = W1.T
    w2t = jnp.pad(W2, ((0, 0), (0, f - c))).T
    b1c = b1.reshape(hdim, 1)
    b2c = jnp.pad(b2, (0, f - c)).reshape(f, 1)

    ys_spec = pl.BlockSpec((f, bs), lambda i, k: (0, k))
    yd_spec = pl.BlockSpec((f, bd), lambda i, k: (0, i))
    d8_spec = pl.BlockSpec((8, bd), lambda i, k: (0, i))
    full = lambda shape: pl.BlockSpec(shape, lambda i, k: (0, 0))
    out_spec = pl.BlockSpec((f, bd), lambda i, k: (0, i))
    acc = pltpu.VMEM((f, bd), fp)

    y2t = pl.pallas_call(
        functools.partial(_layer1_body, n=n, bd=bd, nk=nk),
        grid=(nd, nk),
        in_specs=[a_spec, ys_spec, yd_spec, d8_spec,
                  full((hdim, f)), full((hdim, 1)), full((f, hdim)),
                  pl.BlockSpec(memory_space=pltpu.SMEM)],
        out_specs=out_spec,
        out_shape=jax.ShapeDtypeStruct((f, npad), bf),
        scratch_shapes=[acc],
        compiler_params=params,
    )(ab, y1t, y1t, dinv8, w1t, b1c, w2t, prelu_a.reshape(1))

    outt = pl.pallas_call(
        functools.partial(_layer2_body, c=c, nk=nk),
        grid=(nd, nk),
        in_specs=[a_spec, ys_spec, yd_spec, d8_spec, full((f, 1))],
        out_specs=out_spec,
        out_shape=jax.ShapeDtypeStruct((f, npad), fp),
        scratch_shapes=[acc],
        compiler_params=params,
    )(ab, y2t, y2t, dinv8, b2c)

    return outt[:c, :n].T


# speedup vs baseline: 1.5330x; 1.0037x over previous
import functools

import jax
import jax.numpy as jnp
from jax.experimental import pallas as pl
from jax.experimental.pallas import tpu as pltpu

_NATIVE = (((1,), (0,)), ((), ()))
_LHS_T = (((0,), (0,)), ((), ()))


def _pack_body(a_ref, cs_ref, ab_ref, *, n, bs):
    k = pl.program_id(1)
    a = a_ref[...]
    rows = jax.lax.broadcasted_iota(jnp.int32, a.shape, 0)
    b = jnp.where(rows < n - k * bs, a, 0.0)
    ab_ref[...] = b.astype(jnp.int8)
    ones8 = jnp.ones((8, bs), jnp.float32)
    s = jax.lax.dot_general(ones8, b, _NATIVE,
                            preferred_element_type=jnp.float32)

    @pl.when(k == 0)
    def _init():
        cs_ref[...] = jnp.zeros_like(cs_ref)

    cs_ref[...] += s


def _dinv_row(cs_ref):
    return jax.lax.rsqrt(cs_ref[:1, :] + 1.0)


def _xpanel(x_ref, cs_ref, base, n):
    xt = jnp.transpose(x_ref[...])
    ys = xt * _dinv_row(cs_ref)
    colg = base + jax.lax.broadcasted_iota(jnp.int32, ys.shape, 1)
    return jnp.where(colg < n, ys, 0.0).astype(jnp.bfloat16)


def _layer1_body(ab_ref, xs_ref, xd_ref, css_ref, csd_ref, w1_ref, b1_ref,
                 w2p_ref, pa_ref, out_ref, acc_ref, *, n, bs, bd, nk):
    i = pl.program_id(0)
    k = pl.program_id(1)
    ys = _xpanel(xs_ref, css_ref, k * bs, n)

    @pl.when(k == 0)
    def _init():
        acc_ref[...] = jnp.zeros_like(acc_ref)

    acc_ref[...] += jax.lax.dot_general(
        ys, ab_ref[...].astype(jnp.bfloat16), _NATIVE,
        preferred_element_type=jnp.float32)

    @pl.when(k == nk - 1)
    def _epilogue():
        yd = _xpanel(xd_ref, csd_ref, i * bd, n)
        vt = acc_ref[...] + yd.astype(jnp.float32)
        dinv = _dinv_row(csd_ref)
        ht = jax.lax.dot_general(w1_ref[...], vt, _LHS_T,
                                 preferred_element_type=jnp.float32)
        ht = ht * dinv + b1_ref[...]
        pa = pa_ref[0]
        ht = jnp.where(ht >= 0, ht, pa * ht)
        y2t = jax.lax.dot_general(w2p_ref[...], ht, _LHS_T,
                                  preferred_element_type=jnp.float32) * dinv
        colg = i * bd + jax.lax.broadcasted_iota(jnp.int32, y2t.shape, 1)
        out_ref[...] = jnp.where(colg < n, y2t, 0.0).astype(jnp.bfloat16)


def _layer2_body(ab_ref, ys_ref, yd_ref, csd_ref, b2_ref, out_ref, acc_ref,
                 *, c, nk):
    k = pl.program_id(1)

    @pl.when(k == 0)
    def _init():
        acc_ref[...] = jnp.zeros_like(acc_ref)

    acc_ref[...] += jax.lax.dot_general(
        ys_ref[...], ab_ref[...].astype(jnp.bfloat16), _NATIVE,
        preferred_element_type=jnp.float32)

    @pl.when(k == nk - 1)
    def _epilogue():
        dinv = _dinv_row(csd_ref)
        ut = dinv * (acc_ref[...] + yd_ref[...].astype(jnp.float32)) + b2_ref[...]
        rowv = jax.lax.broadcasted_iota(jnp.int32, ut.shape, 0) < c
        m = jnp.max(jnp.where(rowv, ut, jnp.float32(-1e30)), axis=0,
                    keepdims=True)
        e = jnp.where(rowv, jnp.exp(ut - m), 0.0)
        lse = jnp.log(jnp.sum(e, axis=0, keepdims=True))
        out_ref[...] = ut - m - lse


def kernel(x, A, W1, b1, prelu_a, W2, b2):
    n, din = x.shape
    hdim = W1.shape[1]
    c = W2.shape[1]
    f = 128
    assert din == f

    npad128 = ((n + 127) // 128) * 128
    if npad128 <= 2560:
        npad = npad128
        bs1 = bd1 = bs = bd = npad
    else:
        npad = ((n + 10239) // 10240) * 10240
        bs1, bd1 = 2048, 2560
        bs, bd = 5120, 2560
    nd1, nk1 = npad // bd1, npad // bs1
    nd, nk = npad // bd, npad // bs

    fp = jnp.float32
    bf = jnp.bfloat16
    params = pltpu.CompilerParams(
        dimension_semantics=("parallel", "arbitrary"))

    a1_spec = pl.BlockSpec((bs1, bd1), lambda i, k: (k, i))
    a_spec = pl.BlockSpec((bs, bd), lambda i, k: (k, i))

    colsum, ab = pl.pallas_call(
        functools.partial(_pack_body, n=n, bs=bs1),
        grid=(nd1, nk1),
        in_specs=[a1_spec],
        out_specs=[pl.BlockSpec((8, bd1), lambda i, k: (0, i)), a1_spec],
        out_shape=[jax.ShapeDtypeStruct((8, npad), fp),
                   jax.ShapeDtypeStruct((npad, npad), jnp.int8)],
        compiler_params=params,
    )(A)

    w2p = jnp.pad(W2, ((0, 0), (0, f - c)))
    b1c = b1.reshape(hdim, 1)
    b2c = jnp.pad(b2, (0, f - c)).reshape(f, 1)

    xs_spec = pl.BlockSpec((bs, f), lambda i, k: (k, 0))
    xd_spec = pl.BlockSpec((bd, f), lambda i, k: (i, 0))
    css_spec = pl.BlockSpec((8, bs), lambda i, k: (0, k))
    csd_spec = pl.BlockSpec((8, bd), lambda i, k: (0, i))
    ys_spec = pl.BlockSpec((f, bs), lambda i, k: (0, k))
    yd_spec = pl.BlockSpec((f, bd), lambda i, k: (0, i))
    full = lambda shape: pl.BlockSpec(shape, lambda i, k: (0, 0))
    out_spec = pl.BlockSpec((f, bd), lambda i, k: (0, i))
    acc = pltpu.VMEM((f, bd), fp)

    y2t = pl.pallas_call(
        functools.partial(_layer1_body, n=n, bs=bs, bd=bd, nk=nk),
        grid=(nd, nk),
        in_specs=[a_spec, xs_spec, xd_spec, css_spec, csd_spec,
                  full((din, hdim)), full((hdim, 1)), full((hdim, f)),
                  pl.BlockSpec(memory_space=pltpu.SMEM)],
        out_specs=out_spec,
        out_shape=jax.ShapeDtypeStruct((f, npad), bf),
        scratch_shapes=[acc],
        compiler_params=params,
    )(ab, x, x, colsum, colsum, W1, b1c, w2p, prelu_a.reshape(1))

    outt = pl.pallas_call(
        functools.partial(_layer2_body, c=c, nk=nk),
        grid=(nd, nk),
        in_specs=[a_spec, ys_spec, yd_spec, csd_spec, full((f, 1))],
        out_specs=out_spec,
        out_shape=jax.ShapeDtypeStruct((f, npad), fp),
        scratch_shapes=[acc],
        compiler_params=params,
    )(ab, y2t, y2t, colsum, b2c)

    return outt[:c, :n].T


# speedup vs baseline: 1.5386x; 1.0037x over previous
import functools

import jax
import jax.numpy as jnp
from jax.experimental import pallas as pl
from jax.experimental.pallas import tpu as pltpu

_NATIVE = (((1,), (0,)), ((), ()))
_LHS_T = (((0,), (0,)), ((), ()))


def _pack_body(a_ref, cs_ref, ab_ref, *, n, bs):
    k = pl.program_id(1)
    a = a_ref[...]
    rows = jax.lax.broadcasted_iota(jnp.int32, a.shape, 0)
    b = jnp.where(rows < n - k * bs, a, 0.0)
    ab_ref[...] = b.astype(jnp.int8)
    ones8 = jnp.ones((8, bs), jnp.float32)
    s = jax.lax.dot_general(ones8, b, _NATIVE,
                            preferred_element_type=jnp.float32)

    @pl.when(k == 0)
    def _init():
        cs_ref[...] = jnp.zeros_like(cs_ref)

    cs_ref[...] += s


def _dinv_row(cs_ref):
    return jax.lax.rsqrt(cs_ref[:1, :] + 1.0)


def _xpanel(x_ref, cs_ref, base, n):
    xt = jnp.transpose(x_ref[...])
    ys = xt * _dinv_row(cs_ref)
    colg = base + jax.lax.broadcasted_iota(jnp.int32, ys.shape, 1)
    return jnp.where(colg < n, ys, 0.0).astype(jnp.bfloat16)


def _layer1_body(ab_ref, xs_ref, xd_ref, css_ref, csd_ref, w1_ref, b1_ref,
                 w2p_ref, pa_ref, out_ref, acc_ref, *, n, bs, bd, nk):
    i = pl.program_id(0)
    k = pl.program_id(1)
    ys = _xpanel(xs_ref, css_ref, k * bs, n)

    @pl.when(k == 0)
    def _init():
        acc_ref[...] = jnp.zeros_like(acc_ref)

    h = ys.shape[1] // 2
    p0 = jax.lax.dot_general(
        ys[:, :h], ab_ref[:h, :].astype(jnp.bfloat16), _NATIVE,
        preferred_element_type=jnp.float32)
    p1 = jax.lax.dot_general(
        ys[:, h:], ab_ref[h:, :].astype(jnp.bfloat16), _NATIVE,
        preferred_element_type=jnp.float32)
    acc_ref[...] += p0 + p1

    @pl.when(k == nk - 1)
    def _epilogue():
        yd = _xpanel(xd_ref, csd_ref, i * bd, n)
        vt = acc_ref[...] + yd.astype(jnp.float32)
        dinv = _dinv_row(csd_ref)
        ht = jax.lax.dot_general(w1_ref[...], vt, _LHS_T,
                                 preferred_element_type=jnp.float32)
        ht = ht * dinv + b1_ref[...]
        pa = pa_ref[0]
        ht = jnp.where(ht >= 0, ht, pa * ht)
        y2t = jax.lax.dot_general(w2p_ref[...], ht, _LHS_T,
                                  preferred_element_type=jnp.float32) * dinv
        colg = i * bd + jax.lax.broadcasted_iota(jnp.int32, y2t.shape, 1)
        out_ref[...] = jnp.where(colg < n, y2t, 0.0).astype(jnp.bfloat16)


def _layer2_body(ab_ref, ys_ref, yd_ref, csd_ref, b2_ref, out_ref, acc_ref,
                 *, c, nk):
    k = pl.program_id(1)

    @pl.when(k == 0)
    def _init():
        acc_ref[...] = jnp.zeros_like(acc_ref)

    h = ys_ref.shape[1] // 2
    p0 = jax.lax.dot_general(
        ys_ref[:, :h], ab_ref[:h, :].astype(jnp.bfloat16), _NATIVE,
        preferred_element_type=jnp.float32)
    p1 = jax.lax.dot_general(
        ys_ref[:, h:], ab_ref[h:, :].astype(jnp.bfloat16), _NATIVE,
        preferred_element_type=jnp.float32)
    acc_ref[...] += p0 + p1

    @pl.when(k == nk - 1)
    def _epilogue():
        dinv = _dinv_row(csd_ref)
        ut = dinv * (acc_ref[...] + yd_ref[...].astype(jnp.float32)) + b2_ref[...]
        rowv = jax.lax.broadcasted_iota(jnp.int32, ut.shape, 0) < c
        m = jnp.max(jnp.where(rowv, ut, jnp.float32(-1e30)), axis=0,
                    keepdims=True)
        e = jnp.where(rowv, jnp.exp(ut - m), 0.0)
        lse = jnp.log(jnp.sum(e, axis=0, keepdims=True))
        out_ref[...] = ut - m - lse


def kernel(x, A, W1, b1, prelu_a, W2, b2):
    n, din = x.shape
    hdim = W1.shape[1]
    c = W2.shape[1]
    f = 128
    assert din == f

    npad128 = ((n + 127) // 128) * 128
    if npad128 <= 2560:
        npad = npad128
        bs1 = bd1 = bs = bd = npad
    else:
        npad = ((n + 10239) // 10240) * 10240
        bs1, bd1 = 2048, 2560
        bs, bd = 5120, 2560
    nd1, nk1 = npad // bd1, npad // bs1
    nd, nk = npad // bd, npad // bs

    fp = jnp.float32
    bf = jnp.bfloat16
    params = pltpu.CompilerParams(
        dimension_semantics=("parallel", "arbitrary"))

    a1_spec = pl.BlockSpec((bs1, bd1), lambda i, k: (k, i))
    a_spec = pl.BlockSpec((bs, bd), lambda i, k: (k, i))

    colsum, ab = pl.pallas_call(
        functools.partial(_pack_body, n=n, bs=bs1),
        grid=(nd1, nk1),
        in_specs=[a1_spec],
        out_specs=[pl.BlockSpec((8, bd1), lambda i, k: (0, i)), a1_spec],
        out_shape=[jax.ShapeDtypeStruct((8, npad), fp),
                   jax.ShapeDtypeStruct((npad, npad), jnp.int8)],
        compiler_params=params,
    )(A)

    w2p = jnp.pad(W2, ((0, 0), (0, f - c)))
    b1c = b1.reshape(hdim, 1)
    b2c = jnp.pad(b2, (0, f - c)).reshape(f, 1)

    xs_spec = pl.BlockSpec((bs, f), lambda i, k: (k, 0))
    xd_spec = pl.BlockSpec((bd, f), lambda i, k: (i, 0))
    css_spec = pl.BlockSpec((8, bs), lambda i, k: (0, k))
    csd_spec = pl.BlockSpec((8, bd), lambda i, k: (0, i))
    ys_spec = pl.BlockSpec((f, bs), lambda i, k: (0, k))
    yd_spec = pl.BlockSpec((f, bd), lambda i, k: (0, i))
    full = lambda shape: pl.BlockSpec(shape, lambda i, k: (0, 0))
    out_spec = pl.BlockSpec((f, bd), lambda i, k: (0, i))
    acc = pltpu.VMEM((f, bd), fp)

    y2t = pl.pallas_call(
        functools.partial(_layer1_body, n=n, bs=bs, bd=bd, nk=nk),
        grid=(nd, nk),
        in_specs=[a_spec, xs_spec, xd_spec, css_spec, csd_spec,
                  full((din, hdim)), full((hdim, 1)), full((hdim, f)),
                  pl.BlockSpec(memory_space=pltpu.SMEM)],
        out_specs=out_spec,
        out_shape=jax.ShapeDtypeStruct((f, npad), bf),
        scratch_shapes=[acc],
        compiler_params=params,
    )(ab, x, x, colsum, colsum, W1, b1c, w2p, prelu_a.reshape(1))

    outt = pl.pallas_call(
        functools.partial(_layer2_body, c=c, nk=nk),
        grid=(nd, nk),
        in_specs=[a_spec, ys_spec, yd_spec, csd_spec, full((f, 1))],
        out_specs=out_spec,
        out_shape=jax.ShapeDtypeStruct((f, npad), fp),
        scratch_shapes=[acc],
        compiler_params=params,
    )(ab, y2t, y2t, colsum, b2c)

    return outt[:c, :n].T
